# baseline (speedup 1.0000x reference)
import jax
import jax.numpy as jnp
from jax import lax
from jax.experimental import pallas as pl
from jax.experimental.pallas import tpu as pltpu
from jax.experimental.pallas import tpu_sc as plsc

L = 4096
D = 2048
H = 4096
CODE = 1024
K = 100
KP = 128

NW = 32

NEG = -3.4e38
XOR = 0x7FFFFFFF
INT_MIN = -2147483648


def _bisect_kernel(x_ref, tkey_ref, m_ref):
    x = x_ref[...]
    bu = lax.bitcast_convert_type(x, jnp.int32)
    ks = jnp.where(bu >= 0, bu, bu ^ jnp.int32(XOR))
    cnt0 = jnp.sum((ks >= 0).astype(jnp.int32), axis=1, keepdims=True)
    p = jnp.where(cnt0 >= K, jnp.int32(0), jnp.int32(INT_MIN))
    for b in range(30, -1, -1):
        cand = p + jnp.int32(1 << b)
        cnt = jnp.sum((ks >= cand).astype(jnp.int32), axis=1, keepdims=True)
        p = jnp.where(cnt >= K, cand, p)
    a = jnp.sum((ks > p).astype(jnp.int32), axis=1, keepdims=True)
    tkey_ref[...] = p
    m_ref[...] = K - a


def _make_compact(nrows):
    rpw = nrows // NW

    def body(x_hbm, tkey_hbm, m_hbm, vals_hbm, idx_hbm,
             rowbuf, tkbuf, mbuf, ovals, oidx):
        wid = lax.axis_index("s") * 2 + lax.axis_index("c")
        base = wid * rpw
        pltpu.sync_copy(tkey_hbm.at[pl.ds(base, rpw)], tkbuf)
        pltpu.sync_copy(m_hbm.at[pl.ds(base, rpw)], mbuf)
        ch = 32
        nch = rpw // ch
        iota16 = lax.broadcasted_iota(jnp.int32, (16,), 0)

        def chunk_body(c, _):
            pltpu.sync_copy(x_hbm.at[pl.ds((base + c * ch) * D, ch * D)],
                            rowbuf)

            def row_body(r, _):
                gr = c * ch + r
                roff = r * D
                obase = gr * KP
                grv = jnp.full((16,), gr, jnp.int32)
                tv = plsc.load_gather(tkbuf, [grv])
                av = K - plsc.load_gather(mbuf, [grv])
                ovals[pl.ds(obase + 96, 16)] = jnp.full((16,), NEG,
                                                        jnp.float32)
                ovals[pl.ds(obase + 112, 16)] = jnp.full((16,), NEG,
                                                         jnp.float32)
                oidx[pl.ds(obase + 96, 16)] = iota16 + 4096
                oidx[pl.ds(obase + 112, 16)] = iota16 + 4112

                cg0 = jnp.full((16,), obase, jnp.int32)
                ce0 = cg0 + av
                limit = obase + K

                def scan_vregs(j, carries):
                    cg, ce = carries
                    v = rowbuf[pl.ds(roff + j * 16, 16)]
                    bu = plsc.bitcast(v, jnp.int32)
                    ks = jnp.where(bu >= 0, bu, bu ^ jnp.int32(XOR))
                    gt = ks > tv
                    eq = ks == tv
                    iv = iota16 + j * 16
                    gti = gt.astype(jnp.int32)
                    eqi = eq.astype(jnp.int32)
                    packed = gti + (eqi << 8)
                    ex = plsc.cumsum(packed) - packed
                    tgt = cg + (ex & 0xFF)
                    plsc.store_scatter(ovals, [tgt], v, mask=gt)
                    plsc.store_scatter(oidx, [tgt], iv, mask=gt)
                    pose = ce + (ex >> 8)
                    keep = eq & (pose < limit)
                    plsc.store_scatter(ovals, [pose], v, mask=keep)
                    plsc.store_scatter(oidx, [pose], iv, mask=keep)
                    return (cg + plsc.all_reduce_population_count(gt),
                            ce + plsc.all_reduce_population_count(eq))

                lax.fori_loop(0, KP, scan_vregs, (cg0, ce0), unroll=8)
                return 0

            lax.fori_loop(0, ch, row_body, 0)
            return 0

        lax.fori_loop(0, nch, chunk_body, 0)
        pltpu.sync_copy(ovals, vals_hbm.at[pl.ds(base * KP, rpw * KP)])
        pltpu.sync_copy(oidx, idx_hbm.at[pl.ds(base * KP, rpw * KP)])

    return body, rpw


def _make_scatter(nrows):
    rpw = nrows // NW

    def body(x_hbm, ci_hbm, rk_hbm, e_hbm, simu_hbm,
             cibuf, rkbuf, ebuf, rows):
        wid = lax.axis_index("s") * 2 + lax.axis_index("c")
        base = wid * rpw
        pltpu.sync_copy(ci_hbm.at[pl.ds(base * KP, rpw * KP)], cibuf)
        pltpu.sync_copy(rk_hbm.at[pl.ds(base * KP, rpw * KP)], rkbuf)
        pltpu.sync_copy(e_hbm.at[pl.ds(base * KP, rpw * KP)], ebuf)
        ch = 32
        nch = rpw // ch

        def chunk_body(c, _):
            row0 = base + c * ch
            pltpu.sync_copy(x_hbm.at[pl.ds(row0 * D, ch * D)], rows)

            def row_body(r, _):
                roff = r * D
                ebase = (c * ch + r) * KP
                for j in range(8):
                    rv = rkbuf[pl.ds(ebase + j * 16, 16)]
                    keep = rv < K
                    ev = plsc.load_gather(ebuf, [rv + ebase])
                    iv = cibuf[pl.ds(ebase + j * 16, 16)]
                    iv = jnp.where(keep, iv + roff, 0)
                    plsc.store_scatter(rows, [iv], ev, mask=keep)
                return 0

            lax.fori_loop(0, ch, row_body, 0, unroll=8)
            pltpu.sync_copy(rows, simu_hbm.at[pl.ds(row0 * D, ch * D)])
            return 0

        lax.fori_loop(0, nch, chunk_body, 0)

    return body, rpw


def _rank_mlp_kernel(cv_ref, w_in_ref, b_in_ref, code_ref, w_code_ref,
                     b_code_ref, w_et_ref, w_eb_ref, b_enc_ref,
                     e_ref, rank_ref, cvec):
    i = pl.program_id(0)

    @pl.when(i == 0)
    def _():
        ce = jnp.maximum(
            jnp.dot(code_ref[...], w_code_ref[...],
                    preferred_element_type=jnp.float32) + b_code_ref[...], 0.0)
        cvec[...] = jnp.dot(ce, w_eb_ref[...],
                            preferred_element_type=jnp.float32) + b_enc_ref[...]

    v = cv_ref[...]
    br = v.shape[0]
    va = v[:, :, None]
    vb = v[:, None, :]
    ii3 = lax.broadcasted_iota(jnp.int32, (br, KP, KP), 1)
    kio3 = lax.broadcasted_iota(jnp.int32, (br, KP, KP), 2)
    cmp = (va > vb) | ((va == vb) & (ii3 < kio3))
    rank = jnp.sum(cmp.astype(jnp.int32), axis=1)
    oneh = rank[:, :, None] == kio3
    sv = jnp.sum(jnp.where(oneh, va, 0.0), axis=1)
    kio2 = lax.broadcasted_iota(jnp.int32, (br, KP), 1)
    sv = jnp.where(kio2 < K, sv, 0.0)
    rank_ref[...] = rank
    h = jnp.maximum(
        jnp.dot(sv, w_in_ref[...],
                preferred_element_type=jnp.float32) + b_in_ref[...], 0.0)
    e_ref[...] = jnp.dot(h, w_et_ref[...],
                         preferred_element_type=jnp.float32) + cvec[...]


def _sample_kernel(x_ref, g_ref, wdec_ref, acc_ref, acc):
    i = pl.program_id(0)
    n = pl.num_programs(0)
    br = x_ref.shape[0]
    lane = lax.broadcasted_iota(jnp.int32, (br, D), 1)

    @pl.when(i == 0)
    def _():
        acc[...] = jnp.zeros_like(acc)

    y = x_ref[...] + g_ref[...]
    m2 = jnp.max(y, axis=1, keepdims=True)
    s = jnp.min(jnp.where(y == m2, lane, D), axis=1, keepdims=True)
    sf = s.astype(jnp.float32)
    acc[...] += jnp.sum(sf * wdec_ref[...], axis=0, keepdims=True)

    @pl.when(i == n - 1)
    def _():
        acc_ref[...] = acc[...]


def _finish_kernel(a0_ref, a1_ref, bdec_ref, out_ref):
    o = a0_ref[...] + a1_ref[...] + bdec_ref[...]
    nrm = jnp.sqrt(jnp.sum(o * o))
    out_ref[...] = o / jnp.maximum(nrm, 1e-12)


_SC_PARAMS = pltpu.CompilerParams(needs_layout_passes=False,
                                  use_tc_tiling_on_sc=False)
_SC_MESH = dict(core_axis_name="c", subcore_axis_name="s")


def kernel(input_data, code, W_in, b_in, W_code, b_code, W_enc, b_enc,
           W_dec, b_dec, d_constraint):
    del d_constraint
    f32 = jnp.float32
    i32 = jnp.int32

    w_in_pad = jnp.zeros((KP, H), f32).at[:K].set(W_in)
    w_et = jnp.zeros((H, KP), f32).at[:, :K].set(W_enc[:H])
    w_eb = jnp.zeros((H, KP), f32).at[:, :K].set(W_enc[H:])
    b_enc_pad = jnp.zeros((1, KP), f32).at[0, :K].set(b_enc)
    gumbel = jax.random.gumbel(jax.random.key(1234), (L, D), f32)

    LH = L // 2
    compact_body, _ = _make_compact(LH)
    scatter_body, _ = _make_scatter(LH)
    rpw = LH // NW

    accs = []
    for h in range(2):
        x_h = lax.slice_in_dim(input_data, h * LH, (h + 1) * LH, axis=0)
        g_h = lax.slice_in_dim(gumbel, h * LH, (h + 1) * LH, axis=0)
        wdec_h = lax.slice_in_dim(W_dec, h * LH, (h + 1) * LH, axis=0)
        x_flat = x_h.reshape(LH * D)

        BRB = 512
        tkey, mm = pl.pallas_call(
            _bisect_kernel,
            grid=(LH // BRB,),
            in_specs=[pl.BlockSpec((BRB, D), lambda i: (i, 0))],
            out_specs=[pl.BlockSpec((BRB, 1), lambda i: (i, 0)),
                       pl.BlockSpec((BRB, 1), lambda i: (i, 0))],
            out_shape=[jax.ShapeDtypeStruct((LH, 1), i32),
                       jax.ShapeDtypeStruct((LH, 1), i32)],
        )(x_h)

        cv_flat, ci_flat = pl.kernel(
            compact_body,
            out_type=[jax.ShapeDtypeStruct((LH * KP,), f32),
                      jax.ShapeDtypeStruct((LH * KP,), i32)],
            mesh=plsc.VectorSubcoreMesh(**_SC_MESH),
            scratch_types=[
                pltpu.VMEM((32 * D,), f32),
                pltpu.VMEM((rpw,), i32),
                pltpu.VMEM((rpw,), i32),
                pltpu.VMEM((rpw * KP,), f32),
                pltpu.VMEM((rpw * KP,), i32),
            ],
            compiler_params=_SC_PARAMS,
        )(x_flat, tkey.reshape(LH), mm.reshape(LH))
        cv = cv_flat.reshape(LH, KP)

        BR2 = 128
        e, rank = pl.pallas_call(
            _rank_mlp_kernel,
            grid=(LH // BR2,),
            in_specs=[
                pl.BlockSpec((BR2, KP), lambda i: (i, 0)),
                pl.BlockSpec((KP, H), lambda i: (0, 0)),
                pl.BlockSpec((1, H), lambda i: (0, 0)),
                pl.BlockSpec((1, CODE), lambda i: (0, 0)),
                pl.BlockSpec((CODE, H), lambda i: (0, 0)),
                pl.BlockSpec((1, H), lambda i: (0, 0)),
                pl.BlockSpec((H, KP), lambda i: (0, 0)),
                pl.BlockSpec((H, KP), lambda i: (0, 0)),
                pl.BlockSpec((1, KP), lambda i: (0, 0)),
            ],
            out_specs=[pl.BlockSpec((BR2, KP), lambda i: (i, 0)),
                       pl.BlockSpec((BR2, KP), lambda i: (i, 0))],
            out_shape=[jax.ShapeDtypeStruct((LH, KP), f32),
                       jax.ShapeDtypeStruct((LH, KP), i32)],
            scratch_shapes=[pltpu.VMEM((1, KP), f32)],
        )(cv, w_in_pad, b_in.reshape(1, H), code.reshape(1, CODE), W_code,
          b_code.reshape(1, H), w_et, w_eb, b_enc_pad)

        simu = pl.kernel(
            scatter_body,
            out_type=jax.ShapeDtypeStruct((LH * D,), f32),
            mesh=plsc.VectorSubcoreMesh(**_SC_MESH),
            scratch_types=[
                pltpu.VMEM((rpw * KP,), i32),
                pltpu.VMEM((rpw * KP,), i32),
                pltpu.VMEM((rpw * KP,), f32),
                pltpu.VMEM((32 * D,), f32),
            ],
            compiler_params=_SC_PARAMS,
        )(x_flat, ci_flat, rank.reshape(LH * KP), e.reshape(LH * KP))
        simu = simu.reshape(LH, D)

        BR3 = 512
        acc_h = pl.pallas_call(
            _sample_kernel,
            grid=(LH // BR3,),
            in_specs=[
                pl.BlockSpec((BR3, D), lambda i: (i, 0)),
                pl.BlockSpec((BR3, D), lambda i: (i, 0)),
                pl.BlockSpec((BR3, CODE), lambda i: (i, 0)),
            ],
            out_specs=pl.BlockSpec((1, CODE), lambda i: (0, 0)),
            out_shape=jax.ShapeDtypeStruct((1, CODE), f32),
            scratch_shapes=[pltpu.VMEM((1, CODE), f32)],
        )(simu, g_h, wdec_h)
        accs.append(acc_h)

    out = pl.pallas_call(
        _finish_kernel,
        in_specs=[pl.BlockSpec((1, CODE), lambda: (0, 0)),
                  pl.BlockSpec((1, CODE), lambda: (0, 0)),
                  pl.BlockSpec((1, CODE), lambda: (0, 0))],
        out_specs=pl.BlockSpec((1, CODE), lambda: (0, 0)),
        out_shape=jax.ShapeDtypeStruct((1, CODE), f32),
    )(accs[0], accs[1], b_dec.reshape(1, CODE))

    return out.reshape(CODE)


# speedup vs baseline: 1.0025x; 1.0025x over previous
import jax
import jax.numpy as jnp
from jax import lax
from jax.experimental import pallas as pl
from jax.experimental.pallas import tpu as pltpu
from jax.experimental.pallas import tpu_sc as plsc

L = 4096
D = 2048
H = 4096
CODE = 1024
K = 100
KP = 128

NW = 32

NEG = -3.4e38
XOR = 0x7FFFFFFF
INT_MIN = -2147483648


def _bisect_kernel(x_ref, tkey_ref, m_ref):
    x = x_ref[...]
    bu = lax.bitcast_convert_type(x, jnp.int32)
    ks = jnp.where(bu >= 0, bu, bu ^ jnp.int32(XOR))
    cnt0 = jnp.sum((ks >= 0).astype(jnp.int32), axis=1, keepdims=True)
    p = jnp.where(cnt0 >= K, jnp.int32(0), jnp.int32(INT_MIN))
    for b in range(30, -1, -1):
        cand = p + jnp.int32(1 << b)
        cnt = jnp.sum((ks >= cand).astype(jnp.int32), axis=1, keepdims=True)
        p = jnp.where(cnt >= K, cand, p)
    a = jnp.sum((ks > p).astype(jnp.int32), axis=1, keepdims=True)
    tkey_ref[...] = p
    m_ref[...] = K - a


def _make_compact(nrows):
    rpw = nrows // NW

    def body(x_hbm, tkey_hbm, m_hbm, vals_hbm, idx_hbm,
             rowbuf, tkbuf, mbuf, ovals, oidx):
        wid = lax.axis_index("s") * 2 + lax.axis_index("c")
        base = wid * rpw
        pltpu.sync_copy(tkey_hbm.at[pl.ds(base, rpw)], tkbuf)
        pltpu.sync_copy(m_hbm.at[pl.ds(base, rpw)], mbuf)
        ch = 16
        nch = rpw // ch
        iota16 = lax.broadcasted_iota(jnp.int32, (16,), 0)

        def chunk_body(c, _):
            pltpu.sync_copy(x_hbm.at[pl.ds((base + c * ch) * D, ch * D)],
                            rowbuf)

            def row_body(r, _):
                gr = c * ch + r
                roff = r * D
                obase = gr * KP
                grv = jnp.full((16,), gr, jnp.int32)
                tv = plsc.load_gather(tkbuf, [grv])
                av = K - plsc.load_gather(mbuf, [grv])
                ovals[pl.ds(obase + 96, 16)] = jnp.full((16,), NEG,
                                                        jnp.float32)
                ovals[pl.ds(obase + 112, 16)] = jnp.full((16,), NEG,
                                                         jnp.float32)
                oidx[pl.ds(obase + 96, 16)] = iota16 + 4096
                oidx[pl.ds(obase + 112, 16)] = iota16 + 4112

                cg0 = jnp.full((16,), obase, jnp.int32)
                ce0 = cg0 + av
                limit = obase + K

                def scan_vregs(j, carries):
                    cg, ce = carries
                    v = rowbuf[pl.ds(roff + j * 16, 16)]
                    bu = plsc.bitcast(v, jnp.int32)
                    ks = jnp.where(bu >= 0, bu, bu ^ jnp.int32(XOR))
                    gt = ks > tv
                    eq = ks == tv
                    iv = iota16 + j * 16
                    gti = gt.astype(jnp.int32)
                    eqi = eq.astype(jnp.int32)
                    packed = gti + (eqi << 8)
                    ex = plsc.cumsum(packed) - packed
                    tgt = cg + (ex & 0xFF)
                    plsc.store_scatter(ovals, [tgt], v, mask=gt)
                    plsc.store_scatter(oidx, [tgt], iv, mask=gt)
                    pose = ce + (ex >> 8)
                    keep = eq & (pose < limit)
                    plsc.store_scatter(ovals, [pose], v, mask=keep)
                    plsc.store_scatter(oidx, [pose], iv, mask=keep)
                    return (cg + plsc.all_reduce_population_count(gt),
                            ce + plsc.all_reduce_population_count(eq))

                lax.fori_loop(0, KP, scan_vregs, (cg0, ce0), unroll=8)
                return 0

            lax.fori_loop(0, ch, row_body, 0)
            return 0

        lax.fori_loop(0, nch, chunk_body, 0)
        pltpu.sync_copy(ovals, vals_hbm.at[pl.ds(base * KP, rpw * KP)])
        pltpu.sync_copy(oidx, idx_hbm.at[pl.ds(base * KP, rpw * KP)])

    return body, rpw


def _make_scatter(nrows):
    rpw = nrows // NW

    def body(x_hbm, ci_hbm, rk_hbm, e_hbm, simu_hbm,
             cibuf, rkbuf, ebuf, rows):
        wid = lax.axis_index("s") * 2 + lax.axis_index("c")
        base = wid * rpw
        pltpu.sync_copy(ci_hbm.at[pl.ds(base * KP, rpw * KP)], cibuf)
        pltpu.sync_copy(rk_hbm.at[pl.ds(base * KP, rpw * KP)], rkbuf)
        pltpu.sync_copy(e_hbm.at[pl.ds(base * KP, rpw * KP)], ebuf)
        ch = 32
        nch = rpw // ch

        def chunk_body(c, _):
            row0 = base + c * ch
            pltpu.sync_copy(x_hbm.at[pl.ds(row0 * D, ch * D)], rows)

            def row_body(r, _):
                roff = r * D
                ebase = (c * ch + r) * KP
                for j in range(8):
                    rv = rkbuf[pl.ds(ebase + j * 16, 16)]
                    keep = rv < K
                    ev = plsc.load_gather(ebuf, [rv + ebase])
                    iv = cibuf[pl.ds(ebase + j * 16, 16)]
                    iv = jnp.where(keep, iv + roff, 0)
                    plsc.store_scatter(rows, [iv], ev, mask=keep)
                return 0

            lax.fori_loop(0, ch, row_body, 0, unroll=8)
            pltpu.sync_copy(rows, simu_hbm.at[pl.ds(row0 * D, ch * D)])
            return 0

        lax.fori_loop(0, nch, chunk_body, 0)

    return body, rpw


def _rank_mlp_kernel(cv_ref, w_in_ref, b_in_ref, code_ref, w_code_ref,
                     b_code_ref, w_et_ref, w_eb_ref, b_enc_ref,
                     e_ref, rank_ref, cvec):
    i = pl.program_id(0)

    @pl.when(i == 0)
    def _():
        ce = jnp.maximum(
            jnp.dot(code_ref[...], w_code_ref[...],
                    preferred_element_type=jnp.float32) + b_code_ref[...], 0.0)
        cvec[...] = jnp.dot(ce, w_eb_ref[...],
                            preferred_element_type=jnp.float32) + b_enc_ref[...]

    v = cv_ref[...]
    br = v.shape[0]
    va = v[:, :, None]
    vb = v[:, None, :]
    ii3 = lax.broadcasted_iota(jnp.int32, (br, KP, KP), 1)
    kio3 = lax.broadcasted_iota(jnp.int32, (br, KP, KP), 2)
    cmp = (va > vb) | ((va == vb) & (ii3 < kio3))
    rank = jnp.sum(cmp.astype(jnp.int32), axis=1)
    oneh = rank[:, :, None] == kio3
    sv = jnp.sum(jnp.where(oneh, va, 0.0), axis=1)
    kio2 = lax.broadcasted_iota(jnp.int32, (br, KP), 1)
    sv = jnp.where(kio2 < K, sv, 0.0)
    rank_ref[...] = rank
    h = jnp.maximum(
        jnp.dot(sv, w_in_ref[...],
                preferred_element_type=jnp.float32) + b_in_ref[...], 0.0)
    e_ref[...] = jnp.dot(h, w_et_ref[...],
                         preferred_element_type=jnp.float32) + cvec[...]


def _sample_kernel(x_ref, g_ref, wdec_ref, acc_ref, acc):
    i = pl.program_id(0)
    n = pl.num_programs(0)
    br = x_ref.shape[0]
    lane = lax.broadcasted_iota(jnp.int32, (br, D), 1)

    @pl.when(i == 0)
    def _():
        acc[...] = jnp.zeros_like(acc)

    y = x_ref[...] + g_ref[...]
    m2 = jnp.max(y, axis=1, keepdims=True)
    s = jnp.min(jnp.where(y == m2, lane, D), axis=1, keepdims=True)
    sf = s.astype(jnp.float32)
    acc[...] += jnp.sum(sf * wdec_ref[...], axis=0, keepdims=True)

    @pl.when(i == n - 1)
    def _():
        acc_ref[...] = acc[...]


def _finish_kernel(a0_ref, a1_ref, bdec_ref, out_ref):
    o = a0_ref[...] + a1_ref[...] + bdec_ref[...]
    nrm = jnp.sqrt(jnp.sum(o * o))
    out_ref[...] = o / jnp.maximum(nrm, 1e-12)


_SC_PARAMS = pltpu.CompilerParams(needs_layout_passes=False,
                                  use_tc_tiling_on_sc=False)
_SC_MESH = dict(core_axis_name="c", subcore_axis_name="s")


def kernel(input_data, code, W_in, b_in, W_code, b_code, W_enc, b_enc,
           W_dec, b_dec, d_constraint):
    del d_constraint
    f32 = jnp.float32
    i32 = jnp.int32

    w_in_pad = jnp.zeros((KP, H), f32).at[:K].set(W_in)
    w_et = jnp.zeros((H, KP), f32).at[:, :K].set(W_enc[:H])
    w_eb = jnp.zeros((H, KP), f32).at[:, :K].set(W_enc[H:])
    b_enc_pad = jnp.zeros((1, KP), f32).at[0, :K].set(b_enc)
    gumbel = jax.random.gumbel(jax.random.key(1234), (L, D), f32)

    LH = L // 2
    compact_body, _ = _make_compact(LH)
    scatter_body, _ = _make_scatter(LH)
    rpw = LH // NW

    accs = []
    for h in range(2):
        x_h = lax.slice_in_dim(input_data, h * LH, (h + 1) * LH, axis=0)
        g_h = lax.slice_in_dim(gumbel, h * LH, (h + 1) * LH, axis=0)
        wdec_h = lax.slice_in_dim(W_dec, h * LH, (h + 1) * LH, axis=0)
        x_flat = x_h.reshape(LH * D)

        BRB = 512
        tkey, mm = pl.pallas_call(
            _bisect_kernel,
            grid=(LH // BRB,),
            in_specs=[pl.BlockSpec((BRB, D), lambda i: (i, 0))],
            out_specs=[pl.BlockSpec((BRB, 1), lambda i: (i, 0)),
                       pl.BlockSpec((BRB, 1), lambda i: (i, 0))],
            out_shape=[jax.ShapeDtypeStruct((LH, 1), i32),
                       jax.ShapeDtypeStruct((LH, 1), i32)],
        )(x_h)

        cv_flat, ci_flat = pl.kernel(
            compact_body,
            out_type=[jax.ShapeDtypeStruct((LH * KP,), f32),
                      jax.ShapeDtypeStruct((LH * KP,), i32)],
            mesh=plsc.VectorSubcoreMesh(**_SC_MESH),
            scratch_types=[
                pltpu.VMEM((16 * D,), f32),
                pltpu.VMEM((rpw,), i32),
                pltpu.VMEM((rpw,), i32),
                pltpu.VMEM((rpw * KP,), f32),
                pltpu.VMEM((rpw * KP,), i32),
            ],
            compiler_params=_SC_PARAMS,
        )(x_flat, tkey.reshape(LH), mm.reshape(LH))
        cv = cv_flat.reshape(LH, KP)

        BR2 = 128
        e, rank = pl.pallas_call(
            _rank_mlp_kernel,
            grid=(LH // BR2,),
            in_specs=[
                pl.BlockSpec((BR2, KP), lambda i: (i, 0)),
                pl.BlockSpec((KP, H), lambda i: (0, 0)),
                pl.BlockSpec((1, H), lambda i: (0, 0)),
                pl.BlockSpec((1, CODE), lambda i: (0, 0)),
                pl.BlockSpec((CODE, H), lambda i: (0, 0)),
                pl.BlockSpec((1, H), lambda i: (0, 0)),
                pl.BlockSpec((H, KP), lambda i: (0, 0)),
                pl.BlockSpec((H, KP), lambda i: (0, 0)),
                pl.BlockSpec((1, KP), lambda i: (0, 0)),
            ],
            out_specs=[pl.BlockSpec((BR2, KP), lambda i: (i, 0)),
                       pl.BlockSpec((BR2, KP), lambda i: (i, 0))],
            out_shape=[jax.ShapeDtypeStruct((LH, KP), f32),
                       jax.ShapeDtypeStruct((LH, KP), i32)],
            scratch_shapes=[pltpu.VMEM((1, KP), f32)],
        )(cv, w_in_pad, b_in.reshape(1, H), code.reshape(1, CODE), W_code,
          b_code.reshape(1, H), w_et, w_eb, b_enc_pad)

        simu = pl.kernel(
            scatter_body,
            out_type=jax.ShapeDtypeStruct((LH * D,), f32),
            mesh=plsc.VectorSubcoreMesh(**_SC_MESH),
            scratch_types=[
                pltpu.VMEM((rpw * KP,), i32),
                pltpu.VMEM((rpw * KP,), i32),
                pltpu.VMEM((rpw * KP,), f32),
                pltpu.VMEM((32 * D,), f32),
            ],
            compiler_params=_SC_PARAMS,
        )(x_flat, ci_flat, rank.reshape(LH * KP), e.reshape(LH * KP))
        simu = simu.reshape(LH, D)

        BR3 = 512
        acc_h = pl.pallas_call(
            _sample_kernel,
            grid=(LH // BR3,),
            in_specs=[
                pl.BlockSpec((BR3, D), lambda i: (i, 0)),
                pl.BlockSpec((BR3, D), lambda i: (i, 0)),
                pl.BlockSpec((BR3, CODE), lambda i: (i, 0)),
            ],
            out_specs=pl.BlockSpec((1, CODE), lambda i: (0, 0)),
            out_shape=jax.ShapeDtypeStruct((1, CODE), f32),
            scratch_shapes=[pltpu.VMEM((1, CODE), f32)],
        )(simu, g_h, wdec_h)
        accs.append(acc_h)

    out = pl.pallas_call(
        _finish_kernel,
        in_specs=[pl.BlockSpec((1, CODE), lambda: (0, 0)),
                  pl.BlockSpec((1, CODE), lambda: (0, 0)),
                  pl.BlockSpec((1, CODE), lambda: (0, 0))],
        out_specs=pl.BlockSpec((1, CODE), lambda: (0, 0)),
        out_shape=jax.ShapeDtypeStruct((1, CODE), f32),
    )(accs[0], accs[1], b_dec.reshape(1, CODE))

    return out.reshape(CODE)
